# Initial kernel scaffold; baseline (speedup 1.0000x reference)
import jax
import jax.numpy as jnp
from jax.experimental import pallas as pl


def kernel(x, params, edge_index, edge_index_t, batch):
    raise NotImplementedError("write your pallas kernel here")


# speedup vs baseline: 5.3936x; 5.3936x over previous
import functools

import jax
import jax.numpy as jnp
from jax import lax
from jax.experimental import pallas as pl
from jax.experimental.pallas import tpu as pltpu
from jax.experimental.pallas import tpu_sc as plsc

N = 10000
E = 320000
D = 128
H = 128
Z = 64
G = 64

NC = 2
NS = 16
EPT = E // NS
C = 80
KCH = 25
NBLK = EPT // (C * KCH)
NP = 10240
RPT = NP // NS


def _agg_body(feat, srcs, dsts, out, idx_s, idx_d, rows, acc, sem):
    c = lax.axis_index("c")
    s = lax.axis_index("s")

    def zrow(i, carry):
        for j in range(H // 16):
            rows[i, pl.ds(16 * j, 16)] = jnp.zeros((16,), jnp.float32)
        return carry

    lax.fori_loop(0, C, zrow, 0)

    def zchunk(k, carry):
        pltpu.sync_copy(rows, acc.at[pl.ds(s * RPT + k * C, C)])
        return carry

    lax.fori_loop(0, RPT // C, zchunk, 0)
    plsc.subcore_barrier()

    def block(kk, carry):
        pltpu.sync_copy(srcs.at[c, s, kk], idx_s)
        pltpu.sync_copy(dsts.at[c, s, kk], idx_d)

        def step(k, carry2):
            pltpu.async_copy(feat.at[idx_s.at[k]], rows, sem).wait()
            pltpu.sync_copy(rows, acc.at[idx_d.at[k]], add=True)
            return carry2

        return lax.fori_loop(0, KCH, step, carry)

    lax.fori_loop(0, NBLK, block, 0)
    plsc.subcore_barrier()

    pltpu.sync_copy(acc.at[pl.ds(s * RPT, RPT)], out.at[c, pl.ds(s * RPT, RPT)])


@functools.cache
def _make_agg_kernel():
    return functools.partial(
        pl.kernel,
        out_type=jax.ShapeDtypeStruct((2, NP, H), jnp.float32),
        mesh=plsc.VectorSubcoreMesh(core_axis_name="c", subcore_axis_name="s"),
        scratch_types=[
            pltpu.VMEM((KCH, C), jnp.int32),
            pltpu.VMEM((KCH, C), jnp.int32),
            pltpu.VMEM((C, H), jnp.float32),
            pltpu.VMEM_SHARED((NP, H), jnp.float32),
            pltpu.SemaphoreType.DMA,
        ],
    )(_agg_body)


def _gin_mlp(f, a, w1, b1, g, bb, w2, b2, final_relu):
    h = f + a
    h = jnp.dot(h, w1, preferred_element_type=jnp.float32) + b1
    m = jnp.mean(h, axis=0, keepdims=True)
    v = jnp.mean((h - m) ** 2, axis=0, keepdims=True)
    h = (h - m) * lax.rsqrt(v + 1e-5) * g + bb
    h = jnp.maximum(h, 0.0)
    h = jnp.dot(h, w2, preferred_element_type=jnp.float32) + b2
    if final_relu:
        h = jnp.maximum(h, 0.0)
    return h


def _dense12_body(feat, agg, w1, b1, g, bb, w2, b2, fwa, fwb, fb, out):
    f = feat[...]
    ha = _gin_mlp(f, agg[0, :N], w1[...], b1[...], g[...], bb[...], w2[...], b2[...], True)
    hb = _gin_mlp(f, agg[1, :N], w1[...], b1[...], g[...], bb[...], w2[...], b2[...], True)
    h = (jnp.dot(ha, fwa[...], preferred_element_type=jnp.float32)
         + jnp.dot(hb, fwb[...], preferred_element_type=jnp.float32) + fb[...])
    out[...] = jnp.maximum(h, 0.0)


_dense12 = pl.pallas_call(
    _dense12_body,
    out_shape=jax.ShapeDtypeStruct((N, H), jnp.float32),
)


def _dense3_body(h1r, h2r, agg, batch, w1, b1, g, bb, w2, b2,
                 f3a, f3b, f3bb, f4a, f4b, f4c, f4bb, f5a, f5b, f5c, f5bb,
                 zg_out, h_out):
    h1 = h1r[...]
    h2 = h2r[...]
    ha = _gin_mlp(h2, agg[0, :N], w1[...], b1[...], g[...], bb[...], w2[...], b2[...], False)
    hb = _gin_mlp(h2, agg[1, :N], w1[...], b1[...], g[...], bb[...], w2[...], b2[...], False)
    h3 = jnp.maximum(
        jnp.dot(ha, f3a[...], preferred_element_type=jnp.float32)
        + jnp.dot(hb, f3b[...], preferred_element_type=jnp.float32) + f3bb[...], 0.0)

    sel = (lax.broadcasted_iota(jnp.int32, (G, N), 0) == batch[...]).astype(jnp.float32)
    p1 = jnp.dot(sel, h1, preferred_element_type=jnp.float32)
    p2 = jnp.dot(sel, h2, preferred_element_type=jnp.float32)
    p3 = jnp.dot(sel, h3, preferred_element_type=jnp.float32)
    zg = (jnp.dot(jnp.maximum(p1, 0.0), f4a[...], preferred_element_type=jnp.float32)
          + jnp.dot(jnp.maximum(p2, 0.0), f4b[...], preferred_element_type=jnp.float32)
          + jnp.dot(jnp.maximum(p3, 0.0), f4c[...], preferred_element_type=jnp.float32)
          + f4bb[...])
    zg_out[...] = zg
    h_out[...] = (jnp.dot(h1, f5a[...], preferred_element_type=jnp.float32)
                  + jnp.dot(h2, f5b[...], preferred_element_type=jnp.float32)
                  + jnp.dot(h3, f5c[...], preferred_element_type=jnp.float32)
                  + f5bb[...])


_dense3 = pl.pallas_call(
    _dense3_body,
    out_shape=(jax.ShapeDtypeStruct((G, Z), jnp.float32),
               jax.ShapeDtypeStruct((N, Z), jnp.float32)),
)


def _row(v):
    return v.reshape(1, -1)


def kernel(x, params, edge_index, edge_index_t, batch):
    p = params
    srcs = jnp.stack([edge_index[0], edge_index_t[0]]).reshape(2, NS, NBLK, KCH, C)
    dsts = jnp.stack([edge_index[1], edge_index_t[1]]).reshape(2, NS, NBLK, KCH, C)
    batch2d = batch.reshape(1, N)

    def conv_args(lp, bnp, l2p):
        return (lp["w"], _row(lp["b"]), _row(bnp["g"]), _row(bnp["b"]),
                l2p["w"], _row(l2p["b"]))

    agg = _make_agg_kernel()
    agg1 = agg(x, srcs, dsts)
    h1 = _dense12(x, agg1, *conv_args(p["c1l1"], p["c1bn"], p["c1l2"]),
                  p["fc1"]["w"][:H], p["fc1"]["w"][H:], _row(p["fc1"]["b"]))
    agg2 = agg(h1, srcs, dsts)
    h2 = _dense12(h1, agg2, *conv_args(p["c2l1"], p["c2bn"], p["c2l2"]),
                  p["fc2"]["w"][:H], p["fc2"]["w"][H:], _row(p["fc2"]["b"]))
    agg3 = agg(h2, srcs, dsts)
    zg, h = _dense3(h1, h2, agg3, batch2d,
                    *conv_args(p["c3l1"], p["c3bn"], p["c3l2"]),
                    p["fc3"]["w"][:Z], p["fc3"]["w"][Z:], _row(p["fc3"]["b"]),
                    p["fc4"]["w"][:H], p["fc4"]["w"][H:2 * H], p["fc4"]["w"][2 * H:],
                    _row(p["fc4"]["b"]),
                    p["fc5"]["w"][:H], p["fc5"]["w"][H:2 * H], p["fc5"]["w"][2 * H:],
                    _row(p["fc5"]["b"]))
    return (zg, h)


# speedup vs baseline: 6.5033x; 1.2057x over previous
import functools

import jax
import jax.numpy as jnp
from jax import lax
from jax.experimental import pallas as pl
from jax.experimental.pallas import tpu as pltpu
from jax.experimental.pallas import tpu_sc as plsc

N = 10000
E = 320000
D = 128
H = 128
Z = 64
G = 64

NC = 2
NS = 16
EPT = E // NS
C = 80
KCH = 10
NBLK = EPT // (C * KCH)
NP = 10240
RPT = NP // NS


def _agg_body(feat, srcs, dsts, out, idx_s, idx_d, rows0, rows1, acc, sem0, sem1):
    c = lax.axis_index("c")
    s = lax.axis_index("s")

    def zrow(i, carry):
        for j in range(H // 16):
            rows0[i, pl.ds(16 * j, 16)] = jnp.zeros((16,), jnp.float32)
        return carry

    lax.fori_loop(0, C, zrow, 0)

    def zchunk(k, carry):
        pltpu.sync_copy(rows0, acc.at[pl.ds(s * RPT + k * C, C)])
        return carry

    lax.fori_loop(0, RPT // C, zchunk, 0)
    plsc.subcore_barrier()

    def block(kk, carry):
        pltpu.sync_copy(srcs.at[c, s, kk], idx_s)
        pltpu.sync_copy(dsts.at[c, s, kk], idx_d)
        pltpu.async_copy(feat.at[idx_s.at[0]], rows0, sem0)

        def pair(k2, carry2):
            k = 2 * k2
            pltpu.make_async_copy(feat.at[idx_s.at[k]], rows0, sem0).wait()
            pltpu.async_copy(feat.at[idx_s.at[k + 1]], rows1, sem1)
            pltpu.sync_copy(rows0, acc.at[idx_d.at[k]], add=True)
            pltpu.make_async_copy(feat.at[idx_s.at[k]], rows1, sem1).wait()

            @pl.when(k2 + 1 < KCH // 2)
            def _():
                pltpu.async_copy(feat.at[idx_s.at[k + 2]], rows0, sem0)

            pltpu.sync_copy(rows1, acc.at[idx_d.at[k + 1]], add=True)
            return carry2

        return lax.fori_loop(0, KCH // 2, pair, carry)

    lax.fori_loop(0, NBLK, block, 0)
    plsc.subcore_barrier()

    pltpu.sync_copy(acc.at[pl.ds(s * RPT, RPT)], out.at[c, pl.ds(s * RPT, RPT)])


@functools.cache
def _make_agg_kernel():
    return functools.partial(
        pl.kernel,
        out_type=jax.ShapeDtypeStruct((2, NP, H), jnp.float32),
        mesh=plsc.VectorSubcoreMesh(core_axis_name="c", subcore_axis_name="s"),
        scratch_types=[
            pltpu.VMEM((KCH, C), jnp.int32),
            pltpu.VMEM((KCH, C), jnp.int32),
            pltpu.VMEM((C, H), jnp.float32),
            pltpu.VMEM((C, H), jnp.float32),
            pltpu.VMEM_SHARED((NP, H), jnp.float32),
            pltpu.SemaphoreType.DMA,
            pltpu.SemaphoreType.DMA,
        ],
    )(_agg_body)


def _gin_mlp(f, a, w1, b1, g, bb, w2, b2, final_relu):
    h = f + a
    h = jnp.dot(h, w1, preferred_element_type=jnp.float32) + b1
    m = jnp.mean(h, axis=0, keepdims=True)
    v = jnp.mean((h - m) ** 2, axis=0, keepdims=True)
    h = (h - m) * lax.rsqrt(v + 1e-5) * g + bb
    h = jnp.maximum(h, 0.0)
    h = jnp.dot(h, w2, preferred_element_type=jnp.float32) + b2
    if final_relu:
        h = jnp.maximum(h, 0.0)
    return h


def _dense12_body(feat, agg, w1, b1, g, bb, w2, b2, fwa, fwb, fb, out):
    f = feat[...]
    ha = _gin_mlp(f, agg[0, :N], w1[...], b1[...], g[...], bb[...], w2[...], b2[...], True)
    hb = _gin_mlp(f, agg[1, :N], w1[...], b1[...], g[...], bb[...], w2[...], b2[...], True)
    h = (jnp.dot(ha, fwa[...], preferred_element_type=jnp.float32)
         + jnp.dot(hb, fwb[...], preferred_element_type=jnp.float32) + fb[...])
    out[...] = jnp.maximum(h, 0.0)


_dense12 = pl.pallas_call(
    _dense12_body,
    out_shape=jax.ShapeDtypeStruct((N, H), jnp.float32),
)


def _dense3_body(h1r, h2r, agg, batch, w1, b1, g, bb, w2, b2,
                 f3a, f3b, f3bb, f4a, f4b, f4c, f4bb, f5a, f5b, f5c, f5bb,
                 zg_out, h_out):
    h1 = h1r[...]
    h2 = h2r[...]
    ha = _gin_mlp(h2, agg[0, :N], w1[...], b1[...], g[...], bb[...], w2[...], b2[...], False)
    hb = _gin_mlp(h2, agg[1, :N], w1[...], b1[...], g[...], bb[...], w2[...], b2[...], False)
    h3 = jnp.maximum(
        jnp.dot(ha, f3a[...], preferred_element_type=jnp.float32)
        + jnp.dot(hb, f3b[...], preferred_element_type=jnp.float32) + f3bb[...], 0.0)

    sel = (lax.broadcasted_iota(jnp.int32, (G, N), 0) == batch[...]).astype(jnp.float32)
    p1 = jnp.dot(sel, h1, preferred_element_type=jnp.float32)
    p2 = jnp.dot(sel, h2, preferred_element_type=jnp.float32)
    p3 = jnp.dot(sel, h3, preferred_element_type=jnp.float32)
    zg = (jnp.dot(jnp.maximum(p1, 0.0), f4a[...], preferred_element_type=jnp.float32)
          + jnp.dot(jnp.maximum(p2, 0.0), f4b[...], preferred_element_type=jnp.float32)
          + jnp.dot(jnp.maximum(p3, 0.0), f4c[...], preferred_element_type=jnp.float32)
          + f4bb[...])
    zg_out[...] = zg
    h_out[...] = (jnp.dot(h1, f5a[...], preferred_element_type=jnp.float32)
                  + jnp.dot(h2, f5b[...], preferred_element_type=jnp.float32)
                  + jnp.dot(h3, f5c[...], preferred_element_type=jnp.float32)
                  + f5bb[...])


_dense3 = pl.pallas_call(
    _dense3_body,
    out_shape=(jax.ShapeDtypeStruct((G, Z), jnp.float32),
               jax.ShapeDtypeStruct((N, Z), jnp.float32)),
)


def _row(v):
    return v.reshape(1, -1)


def kernel(x, params, edge_index, edge_index_t, batch):
    p = params
    srcs = jnp.stack([edge_index[0], edge_index_t[0]]).reshape(2, NS, NBLK, KCH, C)
    dsts = jnp.stack([edge_index[1], edge_index_t[1]]).reshape(2, NS, NBLK, KCH, C)
    batch2d = batch.reshape(1, N)

    def conv_args(lp, bnp, l2p):
        return (lp["w"], _row(lp["b"]), _row(bnp["g"]), _row(bnp["b"]),
                l2p["w"], _row(l2p["b"]))

    agg = _make_agg_kernel()
    agg1 = agg(x, srcs, dsts)
    h1 = _dense12(x, agg1, *conv_args(p["c1l1"], p["c1bn"], p["c1l2"]),
                  p["fc1"]["w"][:H], p["fc1"]["w"][H:], _row(p["fc1"]["b"]))
    agg2 = agg(h1, srcs, dsts)
    h2 = _dense12(h1, agg2, *conv_args(p["c2l1"], p["c2bn"], p["c2l2"]),
                  p["fc2"]["w"][:H], p["fc2"]["w"][H:], _row(p["fc2"]["b"]))
    agg3 = agg(h2, srcs, dsts)
    zg, h = _dense3(h1, h2, agg3, batch2d,
                    *conv_args(p["c3l1"], p["c3bn"], p["c3l2"]),
                    p["fc3"]["w"][:Z], p["fc3"]["w"][Z:], _row(p["fc3"]["b"]),
                    p["fc4"]["w"][:H], p["fc4"]["w"][H:2 * H], p["fc4"]["w"][2 * H:],
                    _row(p["fc4"]["b"]),
                    p["fc5"]["w"][:H], p["fc5"]["w"][H:2 * H], p["fc5"]["w"][2 * H:],
                    _row(p["fc5"]["b"]))
    return (zg, h)


# speedup vs baseline: 6.6577x; 1.0237x over previous
import functools

import jax
import jax.numpy as jnp
from jax import lax
from jax.experimental import pallas as pl
from jax.experimental.pallas import tpu as pltpu
from jax.experimental.pallas import tpu_sc as plsc

N = 10000
E = 320000
D = 128
H = 128
Z = 64
G = 64

NC = 2
NS = 16
EPT = E // NS
C = 80
KCH = 10
NBLK = EPT // (C * KCH)
NP = 10240
RPT = NP // NS


def _agg_body(feat, srcs, dsts, out, idx_s, idx_d, rows0, rows1, acc,
              semg0, semg1, sems0, sems1):
    c = lax.axis_index("c")
    s = lax.axis_index("s")

    def zrow(i, carry):
        for j in range(H // 16):
            rows0[i, pl.ds(16 * j, 16)] = jnp.zeros((16,), jnp.float32)
        return carry

    lax.fori_loop(0, C, zrow, 0)

    def zchunk(k, carry):
        pltpu.sync_copy(rows0, acc.at[pl.ds(s * RPT + k * C, C)])
        return carry

    lax.fori_loop(0, RPT // C, zchunk, 0)
    plsc.subcore_barrier()

    def wait_gather(buf, sem):
        pltpu.make_async_copy(feat.at[idx_s.at[0]], buf, sem).wait()

    def wait_scatter(buf, sem):
        pltpu.make_async_copy(buf, acc.at[idx_d.at[0]], sem).wait()

    def block(kk, carry):
        pltpu.sync_copy(srcs.at[c, s, kk], idx_s)
        pltpu.sync_copy(dsts.at[c, s, kk], idx_d)
        pltpu.async_copy(feat.at[idx_s.at[0]], rows0, semg0)
        pltpu.async_copy(feat.at[idx_s.at[1]], rows1, semg1)

        def pair(k2, carry2):
            k = 2 * k2
            wait_gather(rows0, semg0)
            pltpu.async_copy(rows0, acc.at[idx_d.at[k]], sems0, add=True)
            wait_gather(rows1, semg1)
            pltpu.async_copy(rows1, acc.at[idx_d.at[k + 1]], sems1, add=True)

            @pl.when(k + 2 < KCH)
            def _():
                wait_scatter(rows0, sems0)
                pltpu.async_copy(feat.at[idx_s.at[k + 2]], rows0, semg0)
                wait_scatter(rows1, sems1)
                pltpu.async_copy(feat.at[idx_s.at[k + 3]], rows1, semg1)

            return carry2

        out_c = lax.fori_loop(0, KCH // 2, pair, carry)
        wait_scatter(rows0, sems0)
        wait_scatter(rows1, sems1)
        return out_c

    lax.fori_loop(0, NBLK, block, 0)
    plsc.subcore_barrier()

    pltpu.sync_copy(acc.at[pl.ds(s * RPT, RPT)], out.at[c, pl.ds(s * RPT, RPT)])


@functools.cache
def _make_agg_kernel():
    return functools.partial(
        pl.kernel,
        out_type=jax.ShapeDtypeStruct((2, NP, H), jnp.float32),
        mesh=plsc.VectorSubcoreMesh(core_axis_name="c", subcore_axis_name="s"),
        scratch_types=[
            pltpu.VMEM((KCH, C), jnp.int32),
            pltpu.VMEM((KCH, C), jnp.int32),
            pltpu.VMEM((C, H), jnp.float32),
            pltpu.VMEM((C, H), jnp.float32),
            pltpu.VMEM_SHARED((NP, H), jnp.float32),
            pltpu.SemaphoreType.DMA,
            pltpu.SemaphoreType.DMA,
            pltpu.SemaphoreType.DMA,
            pltpu.SemaphoreType.DMA,
        ],
    )(_agg_body)


def _gin_mlp(f, a, w1, b1, g, bb, w2, b2, final_relu):
    h = f + a
    h = jnp.dot(h, w1, preferred_element_type=jnp.float32) + b1
    m = jnp.mean(h, axis=0, keepdims=True)
    v = jnp.mean((h - m) ** 2, axis=0, keepdims=True)
    h = (h - m) * lax.rsqrt(v + 1e-5) * g + bb
    h = jnp.maximum(h, 0.0)
    h = jnp.dot(h, w2, preferred_element_type=jnp.float32) + b2
    if final_relu:
        h = jnp.maximum(h, 0.0)
    return h


def _dense12_body(feat, agg, w1, b1, g, bb, w2, b2, fwa, fwb, fb, out):
    f = feat[...]
    ha = _gin_mlp(f, agg[0, :N], w1[...], b1[...], g[...], bb[...], w2[...], b2[...], True)
    hb = _gin_mlp(f, agg[1, :N], w1[...], b1[...], g[...], bb[...], w2[...], b2[...], True)
    h = (jnp.dot(ha, fwa[...], preferred_element_type=jnp.float32)
         + jnp.dot(hb, fwb[...], preferred_element_type=jnp.float32) + fb[...])
    out[...] = jnp.maximum(h, 0.0)


_dense12 = pl.pallas_call(
    _dense12_body,
    out_shape=jax.ShapeDtypeStruct((N, H), jnp.float32),
)


def _dense3_body(h1r, h2r, agg, batch, w1, b1, g, bb, w2, b2,
                 f3a, f3b, f3bb, f4a, f4b, f4c, f4bb, f5a, f5b, f5c, f5bb,
                 zg_out, h_out):
    h1 = h1r[...]
    h2 = h2r[...]
    ha = _gin_mlp(h2, agg[0, :N], w1[...], b1[...], g[...], bb[...], w2[...], b2[...], False)
    hb = _gin_mlp(h2, agg[1, :N], w1[...], b1[...], g[...], bb[...], w2[...], b2[...], False)
    h3 = jnp.maximum(
        jnp.dot(ha, f3a[...], preferred_element_type=jnp.float32)
        + jnp.dot(hb, f3b[...], preferred_element_type=jnp.float32) + f3bb[...], 0.0)

    sel = (lax.broadcasted_iota(jnp.int32, (G, N), 0) == batch[...]).astype(jnp.float32)
    p1 = jnp.dot(sel, h1, preferred_element_type=jnp.float32)
    p2 = jnp.dot(sel, h2, preferred_element_type=jnp.float32)
    p3 = jnp.dot(sel, h3, preferred_element_type=jnp.float32)
    zg = (jnp.dot(jnp.maximum(p1, 0.0), f4a[...], preferred_element_type=jnp.float32)
          + jnp.dot(jnp.maximum(p2, 0.0), f4b[...], preferred_element_type=jnp.float32)
          + jnp.dot(jnp.maximum(p3, 0.0), f4c[...], preferred_element_type=jnp.float32)
          + f4bb[...])
    zg_out[...] = zg
    h_out[...] = (jnp.dot(h1, f5a[...], preferred_element_type=jnp.float32)
                  + jnp.dot(h2, f5b[...], preferred_element_type=jnp.float32)
                  + jnp.dot(h3, f5c[...], preferred_element_type=jnp.float32)
                  + f5bb[...])


_dense3 = pl.pallas_call(
    _dense3_body,
    out_shape=(jax.ShapeDtypeStruct((G, Z), jnp.float32),
               jax.ShapeDtypeStruct((N, Z), jnp.float32)),
)


def _row(v):
    return v.reshape(1, -1)


def kernel(x, params, edge_index, edge_index_t, batch):
    p = params
    srcs = jnp.stack([edge_index[0], edge_index_t[0]]).reshape(2, NS, NBLK, KCH, C)
    dsts = jnp.stack([edge_index[1], edge_index_t[1]]).reshape(2, NS, NBLK, KCH, C)
    batch2d = batch.reshape(1, N)

    def conv_args(lp, bnp, l2p):
        return (lp["w"], _row(lp["b"]), _row(bnp["g"]), _row(bnp["b"]),
                l2p["w"], _row(l2p["b"]))

    agg = _make_agg_kernel()
    agg1 = agg(x, srcs, dsts)
    h1 = _dense12(x, agg1, *conv_args(p["c1l1"], p["c1bn"], p["c1l2"]),
                  p["fc1"]["w"][:H], p["fc1"]["w"][H:], _row(p["fc1"]["b"]))
    agg2 = agg(h1, srcs, dsts)
    h2 = _dense12(h1, agg2, *conv_args(p["c2l1"], p["c2bn"], p["c2l2"]),
                  p["fc2"]["w"][:H], p["fc2"]["w"][H:], _row(p["fc2"]["b"]))
    agg3 = agg(h2, srcs, dsts)
    zg, h = _dense3(h1, h2, agg3, batch2d,
                    *conv_args(p["c3l1"], p["c3bn"], p["c3l2"]),
                    p["fc3"]["w"][:Z], p["fc3"]["w"][Z:], _row(p["fc3"]["b"]),
                    p["fc4"]["w"][:H], p["fc4"]["w"][H:2 * H], p["fc4"]["w"][2 * H:],
                    _row(p["fc4"]["b"]),
                    p["fc5"]["w"][:H], p["fc5"]["w"][H:2 * H], p["fc5"]["w"][2 * H:],
                    _row(p["fc5"]["b"]))
    return (zg, h)
